# baseline (speedup 1.0000x reference)
import functools

import jax
import jax.numpy as jnp
from jax import lax
from jax.experimental import pallas as pl
from jax.experimental.pallas import tpu as pltpu
from jax.experimental.pallas import tpu_sc as plsc

N = 50000
D = 64
H = 32
BLK = 3584
N_PAD = 50176
E = 800000
CHUNK = 32
NSUB = 16
NCH = 1568
IBLK = 16
NPAIR = NCH // 8
E_PAD = NSUB * NCH * CHUNK
EROWS = E_PAD // CHUNK
RPT = N_PAD // NSUB


def _sc_body(et, taba, tabb, zeros, outa, outb,
             sidx, didx, r0, r1, r2, r3, r4, r5, r6, r7,
             acc, g0, g1, g2, g3, g4, g5, g6, g7, ssem):
    srcs = et.at[0]
    dsts = et.at[1]
    c = lax.axis_index("c")
    s = lax.axis_index("s")
    base = s * RPT
    rows = (r0, r1, r2, r3, r4, r5, r6, r7)
    gsems = (g0, g1, g2, g3, g4, g5, g6, g7)

    pltpu.sync_copy(zeros, acc.at[pl.ds(base, RPT)])

    rbase = s * NCH

    plsc.subcore_barrier()

    def drain4():
        pltpu.make_async_copy(zeros.at[pl.ds(0, 4 * CHUNK)],
                              acc.at[pl.ds(0, 4 * CHUNK)], ssem).wait()

    def edge_loop(tab):
        def pair(k, carry):
            @pl.when((k % 2 == 0) & (k > 0))
            def _():
                drain4()
                drain4()

            @pl.when(k % 2 == 0)
            def _():
                ib = rbase + (k // 2) * IBLK
                pltpu.sync_copy(srcs.at[pl.ds(ib, IBLK)], sidx)
                pltpu.sync_copy(dsts.at[pl.ds(ib, IBLK)], didx)

            jb = (k % 2) * 8
            for half in range(2):
                @pl.when(k % 2 == 1)
                def _():
                    drain4()
                hb = half * 4
                cps = [pltpu.async_copy(tab.at[sidx.at[jb + hb + i]],
                                        rows[hb + i], gsems[hb + i])
                       for i in range(4)]
                for i in range(4):
                    cps[i].wait()
                    pltpu.async_copy(rows[hb + i],
                                     acc.at[didx.at[jb + hb + i]], ssem,
                                     add=True)
            return carry
        lax.fori_loop(0, NPAIR, pair, 0)
        drain4()
        drain4()

    @pl.when(c == 0)
    def _():
        edge_loop(taba)

    @pl.when(c == 1)
    def _():
        edge_loop(tabb)

    plsc.subcore_barrier()

    @pl.when(c == 0)
    def _():
        pltpu.sync_copy(acc.at[pl.ds(base, RPT)], outa.at[pl.ds(base, RPT)])

    @pl.when(c == 1)
    def _():
        pltpu.sync_copy(acc.at[pl.ds(base, RPT)], outb.at[pl.ds(base, RPT)])


@functools.cache
def _make_sc_round():
    return functools.partial(
        pl.kernel,
        out_type=(jax.ShapeDtypeStruct((N_PAD, H), jnp.float32),
                  jax.ShapeDtypeStruct((N_PAD, H), jnp.float32)),
        mesh=plsc.VectorSubcoreMesh(core_axis_name="c", subcore_axis_name="s"),
        scratch_types=(
            [pltpu.VMEM((IBLK, CHUNK), jnp.int32)] * 2
            + [pltpu.VMEM((CHUNK, H), jnp.float32)] * 8
            + [pltpu.VMEM_SHARED((N_PAD, H), jnp.float32)]
            + [pltpu.SemaphoreType.DMA] * 9
        ),
        compiler_params=pltpu.CompilerParams(use_tc_tiling_on_sc=False),
    )(_sc_body)


def _sc_round(et, taba, tabb, zeros):
    return _make_sc_round()(et, taba, tabb, zeros)


def _t0_body(nodes_ref, feat_ref, nemb_ref, wf_ref, bf_ref, oa_ref, ob_ref):
    nid = nodes_ref[...]
    iota = lax.broadcasted_iota(jnp.int32, (BLK, 128), 1)
    onehot = (iota == nid).astype(jnp.float32)
    feat = feat_ref[...]
    nemb = nemb_ref[...]
    wf = wf_ref[...]
    bf = bf_ref[...]
    for out, lo in ((oa_ref, 0), (ob_ref, H)):
        emb = jnp.dot(onehot, nemb[:, lo:lo + H],
                      preferred_element_type=jnp.float32)
        fe = jnp.dot(feat, wf[:, lo:lo + H],
                     preferred_element_type=jnp.float32)
        out[...] = emb + fe + bf[:, lo:lo + H]


def _t_init(nodes2d, featp, nembp, wfp, bf2d):
    grid = (N_PAD // BLK,)
    return pl.pallas_call(
        _t0_body,
        grid=grid,
        in_specs=[
            pl.BlockSpec((BLK, 1), lambda i: (i, 0)),
            pl.BlockSpec((BLK, 16), lambda i: (i, 0)),
            pl.BlockSpec((128, D), lambda i: (0, 0)),
            pl.BlockSpec((16, D), lambda i: (0, 0)),
            pl.BlockSpec((1, D), lambda i: (0, 0)),
        ],
        out_specs=(pl.BlockSpec((BLK, H), lambda i: (i, 0)),
                   pl.BlockSpec((BLK, H), lambda i: (i, 0))),
        out_shape=(jax.ShapeDtypeStruct((N_PAD, H), jnp.float32),
                   jax.ShapeDtypeStruct((N_PAD, H), jnp.float32)),
    )(nodes2d, featp, nembp, wfp, bf2d)


def _t1_body(ea_ref, eb_ref, na_ref, nb_ref, w1_ref, b1_ref, oa_ref, ob_ref):
    xa = ea_ref[...] + na_ref[...]
    xb = eb_ref[...] + nb_ref[...]
    w1 = w1_ref[...]
    b1 = b1_ref[...]
    for out, lo in ((oa_ref, 0), (ob_ref, H)):
        h = (jnp.dot(xa, w1[:H, lo:lo + H], preferred_element_type=jnp.float32)
             + jnp.dot(xb, w1[H:, lo:lo + H],
                       preferred_element_type=jnp.float32))
        out[...] = jnp.maximum(h + b1[:, lo:lo + H], 0.0)


def _t_round(ea, eb, na, nb, w1, b12d):
    grid = (N_PAD // BLK,)
    half = pl.BlockSpec((BLK, H), lambda i: (i, 0))
    return pl.pallas_call(
        _t1_body,
        grid=grid,
        in_specs=[half, half, half, half,
                  pl.BlockSpec((D, D), lambda i: (0, 0)),
                  pl.BlockSpec((1, D), lambda i: (0, 0))],
        out_specs=(half, half),
        out_shape=(jax.ShapeDtypeStruct((N_PAD, H), jnp.float32),
                   jax.ShapeDtypeStruct((N_PAD, H), jnp.float32)),
    )(ea, eb, na, nb, w1, b12d)


def _t2_body(ea_ref, eb_ref, na_ref, nb_ref, w1_ref, b1_ref, p_ref, o_ref):
    xa = ea_ref[...] + na_ref[...]
    xb = eb_ref[...] + nb_ref[...]
    w1 = w1_ref[...]
    b1 = b1_ref[...]
    p = p_ref[...]
    num = jnp.zeros((BLK, 1), jnp.float32)
    en2 = jnp.zeros((BLK, 1), jnp.float32)
    for lo in (0, H):
        h = (jnp.dot(xa, w1[:H, lo:lo + H], preferred_element_type=jnp.float32)
             + jnp.dot(xb, w1[H:, lo:lo + H],
                       preferred_element_type=jnp.float32))
        h = jnp.maximum(h + b1[:, lo:lo + H], 0.0)
        num = num + jnp.dot(h, p[:, lo:lo + H].T,
                            preferred_element_type=jnp.float32)
        en2 = en2 + jnp.sum(h * h, axis=1, keepdims=True)
    en = jnp.sqrt(en2)
    pn = jnp.sqrt(jnp.sum(p * p))
    o_ref[...] = num / (jnp.maximum(en, 1e-8) * jnp.maximum(pn, 1e-8))


def _t_final(ea, eb, na, nb, w1, b12d, p2d):
    grid = (N_PAD // BLK,)
    half = pl.BlockSpec((BLK, H), lambda i: (i, 0))
    return pl.pallas_call(
        _t2_body,
        grid=grid,
        in_specs=[half, half, half, half,
                  pl.BlockSpec((D, D), lambda i: (0, 0)),
                  pl.BlockSpec((1, D), lambda i: (0, 0)),
                  pl.BlockSpec((1, D), lambda i: (0, 0))],
        out_specs=pl.BlockSpec((BLK, 1), lambda i: (i, 0)),
        out_shape=jax.ShapeDtypeStruct((N_PAD, 1), jnp.float32),
    )(ea, eb, na, nb, w1, b12d, p2d)


@jax.jit
def _run(nodes, edges, features, node_emb, W_f, b_f, W1, b1, pattern_emb,
         node_type_id):
    nodes2d = jnp.zeros((N_PAD, 1), jnp.int32).at[:N, 0].set(
        nodes.astype(jnp.int32))
    featp = jnp.pad(features, ((0, N_PAD - N), (0, 6)))
    nembp = jnp.zeros((128, D), jnp.float32).at[:100].set(node_emb)
    wfp = jnp.pad(W_f, ((0, 6), (0, 0)))
    bf2d = b_f.reshape(1, D)
    b12d = b1.reshape(1, D)

    et = jnp.pad(edges.T.astype(jnp.int32), ((0, 0), (0, E_PAD - E)),
                 constant_values=N).reshape(2, EROWS, CHUNK)
    zeros = jnp.zeros((RPT, H), jnp.float32)

    p2d = lax.dynamic_slice_in_dim(pattern_emb, node_type_id, 1, axis=0)

    ea, eb = _t_init(nodes2d, featp, nembp, wfp, bf2d)
    na, nb = _sc_round(et, ea, eb, zeros)
    ea, eb = _t_round(ea, eb, na, nb, W1, b12d)
    na, nb = _sc_round(et, ea, eb, zeros)
    out = _t_final(ea, eb, na, nb, W1, b12d, p2d)
    return out[:N, 0]


def kernel(nodes, edges, features, node_emb, W_f, b_f, W1, b1, pattern_emb,
           node_type_id):
    return _run(nodes, edges, features, node_emb, W_f, b_f, W1, b1,
                pattern_emb, jnp.asarray(node_type_id, jnp.int32))


# speedup vs baseline: 1.3862x; 1.3862x over previous
import functools

import jax
import jax.numpy as jnp
from jax import lax
from jax.experimental import pallas as pl
from jax.experimental.pallas import tpu as pltpu
from jax.experimental.pallas import tpu_sc as plsc

N = 50000
D = 64
H = 32
BLK = 3584
N_PAD = 50176
E = 800000
CHUNK = 64
NSUB = 16
NCH = 784
IBLK = 32
NPAIR = NCH // 8
E_PAD = NSUB * NCH * CHUNK
EROWS = E_PAD // CHUNK
RPT = N_PAD // NSUB


def _sc_body(et, taba, tabb, zeros, outa, outb,
             sidx, didx, r0, r1, r2, r3, r4, r5, r6, r7,
             acc, g0, g1, g2, g3, g4, g5, g6, g7, ssem):
    srcs = et.at[0]
    dsts = et.at[1]
    c = lax.axis_index("c")
    s = lax.axis_index("s")
    base = s * RPT
    rows = (r0, r1, r2, r3, r4, r5, r6, r7)
    gsems = (g0, g1, g2, g3, g4, g5, g6, g7)

    pltpu.sync_copy(zeros, acc.at[pl.ds(base, RPT)])

    rbase = s * NCH

    plsc.subcore_barrier()

    def drain4():
        pltpu.make_async_copy(zeros.at[pl.ds(0, 4 * CHUNK)],
                              acc.at[pl.ds(0, 4 * CHUNK)], ssem).wait()

    def edge_loop(tab):
        def pair(k, carry):
            @pl.when((k % 4 == 0) & (k > 0))
            def _():
                drain4()
                drain4()

            @pl.when(k % 4 == 0)
            def _():
                ib = rbase + (k // 4) * IBLK
                pltpu.sync_copy(srcs.at[pl.ds(ib, IBLK)], sidx)
                pltpu.sync_copy(dsts.at[pl.ds(ib, IBLK)], didx)

            jb = (k % 4) * 8
            for half in range(2):
                @pl.when(k % 4 != 0)
                def _():
                    drain4()
                hb = half * 4
                cps = [pltpu.async_copy(tab.at[sidx.at[jb + hb + i]],
                                        rows[hb + i], gsems[hb + i])
                       for i in range(4)]
                for i in range(4):
                    cps[i].wait()
                    pltpu.async_copy(rows[hb + i],
                                     acc.at[didx.at[jb + hb + i]], ssem,
                                     add=True)
            return carry
        lax.fori_loop(0, NPAIR, pair, 0)
        drain4()
        drain4()

    @pl.when(c == 0)
    def _():
        edge_loop(taba)

    @pl.when(c == 1)
    def _():
        edge_loop(tabb)

    plsc.subcore_barrier()

    @pl.when(c == 0)
    def _():
        pltpu.sync_copy(acc.at[pl.ds(base, RPT)], outa.at[pl.ds(base, RPT)])

    @pl.when(c == 1)
    def _():
        pltpu.sync_copy(acc.at[pl.ds(base, RPT)], outb.at[pl.ds(base, RPT)])


@functools.cache
def _make_sc_round():
    return functools.partial(
        pl.kernel,
        out_type=(jax.ShapeDtypeStruct((N_PAD, H), jnp.float32),
                  jax.ShapeDtypeStruct((N_PAD, H), jnp.float32)),
        mesh=plsc.VectorSubcoreMesh(core_axis_name="c", subcore_axis_name="s"),
        scratch_types=(
            [pltpu.VMEM((IBLK, CHUNK), jnp.int32)] * 2
            + [pltpu.VMEM((CHUNK, H), jnp.float32)] * 8
            + [pltpu.VMEM_SHARED((N_PAD, H), jnp.float32)]
            + [pltpu.SemaphoreType.DMA] * 9
        ),
        compiler_params=pltpu.CompilerParams(use_tc_tiling_on_sc=False),
    )(_sc_body)


def _sc_round(et, taba, tabb, zeros):
    return _make_sc_round()(et, taba, tabb, zeros)


def _t0_body(nodes_ref, feat_ref, nemb_ref, wf_ref, bf_ref, oa_ref, ob_ref):
    nid = nodes_ref[...]
    iota = lax.broadcasted_iota(jnp.int32, (BLK, 128), 1)
    onehot = (iota == nid).astype(jnp.float32)
    feat = feat_ref[...]
    nemb = nemb_ref[...]
    wf = wf_ref[...]
    bf = bf_ref[...]
    for out, lo in ((oa_ref, 0), (ob_ref, H)):
        emb = jnp.dot(onehot, nemb[:, lo:lo + H],
                      preferred_element_type=jnp.float32)
        fe = jnp.dot(feat, wf[:, lo:lo + H],
                     preferred_element_type=jnp.float32)
        out[...] = emb + fe + bf[:, lo:lo + H]


def _t_init(nodes2d, featp, nembp, wfp, bf2d):
    grid = (N_PAD // BLK,)
    return pl.pallas_call(
        _t0_body,
        grid=grid,
        in_specs=[
            pl.BlockSpec((BLK, 1), lambda i: (i, 0)),
            pl.BlockSpec((BLK, 16), lambda i: (i, 0)),
            pl.BlockSpec((128, D), lambda i: (0, 0)),
            pl.BlockSpec((16, D), lambda i: (0, 0)),
            pl.BlockSpec((1, D), lambda i: (0, 0)),
        ],
        out_specs=(pl.BlockSpec((BLK, H), lambda i: (i, 0)),
                   pl.BlockSpec((BLK, H), lambda i: (i, 0))),
        out_shape=(jax.ShapeDtypeStruct((N_PAD, H), jnp.float32),
                   jax.ShapeDtypeStruct((N_PAD, H), jnp.float32)),
    )(nodes2d, featp, nembp, wfp, bf2d)


def _t1_body(ea_ref, eb_ref, na_ref, nb_ref, w1_ref, b1_ref, oa_ref, ob_ref):
    xa = ea_ref[...] + na_ref[...]
    xb = eb_ref[...] + nb_ref[...]
    w1 = w1_ref[...]
    b1 = b1_ref[...]
    for out, lo in ((oa_ref, 0), (ob_ref, H)):
        h = (jnp.dot(xa, w1[:H, lo:lo + H], preferred_element_type=jnp.float32)
             + jnp.dot(xb, w1[H:, lo:lo + H],
                       preferred_element_type=jnp.float32))
        out[...] = jnp.maximum(h + b1[:, lo:lo + H], 0.0)


def _t_round(ea, eb, na, nb, w1, b12d):
    grid = (N_PAD // BLK,)
    half = pl.BlockSpec((BLK, H), lambda i: (i, 0))
    return pl.pallas_call(
        _t1_body,
        grid=grid,
        in_specs=[half, half, half, half,
                  pl.BlockSpec((D, D), lambda i: (0, 0)),
                  pl.BlockSpec((1, D), lambda i: (0, 0))],
        out_specs=(half, half),
        out_shape=(jax.ShapeDtypeStruct((N_PAD, H), jnp.float32),
                   jax.ShapeDtypeStruct((N_PAD, H), jnp.float32)),
    )(ea, eb, na, nb, w1, b12d)


def _t2_body(ea_ref, eb_ref, na_ref, nb_ref, w1_ref, b1_ref, p_ref, o_ref):
    xa = ea_ref[...] + na_ref[...]
    xb = eb_ref[...] + nb_ref[...]
    w1 = w1_ref[...]
    b1 = b1_ref[...]
    p = p_ref[...]
    num = jnp.zeros((BLK, 1), jnp.float32)
    en2 = jnp.zeros((BLK, 1), jnp.float32)
    for lo in (0, H):
        h = (jnp.dot(xa, w1[:H, lo:lo + H], preferred_element_type=jnp.float32)
             + jnp.dot(xb, w1[H:, lo:lo + H],
                       preferred_element_type=jnp.float32))
        h = jnp.maximum(h + b1[:, lo:lo + H], 0.0)
        num = num + jnp.dot(h, p[:, lo:lo + H].T,
                            preferred_element_type=jnp.float32)
        en2 = en2 + jnp.sum(h * h, axis=1, keepdims=True)
    en = jnp.sqrt(en2)
    pn = jnp.sqrt(jnp.sum(p * p))
    o_ref[...] = num / (jnp.maximum(en, 1e-8) * jnp.maximum(pn, 1e-8))


def _t_final(ea, eb, na, nb, w1, b12d, p2d):
    grid = (N_PAD // BLK,)
    half = pl.BlockSpec((BLK, H), lambda i: (i, 0))
    return pl.pallas_call(
        _t2_body,
        grid=grid,
        in_specs=[half, half, half, half,
                  pl.BlockSpec((D, D), lambda i: (0, 0)),
                  pl.BlockSpec((1, D), lambda i: (0, 0)),
                  pl.BlockSpec((1, D), lambda i: (0, 0))],
        out_specs=pl.BlockSpec((BLK, 1), lambda i: (i, 0)),
        out_shape=jax.ShapeDtypeStruct((N_PAD, 1), jnp.float32),
    )(ea, eb, na, nb, w1, b12d, p2d)


@jax.jit
def _run(nodes, edges, features, node_emb, W_f, b_f, W1, b1, pattern_emb,
         node_type_id):
    nodes2d = jnp.zeros((N_PAD, 1), jnp.int32).at[:N, 0].set(
        nodes.astype(jnp.int32))
    featp = jnp.pad(features, ((0, N_PAD - N), (0, 6)))
    nembp = jnp.zeros((128, D), jnp.float32).at[:100].set(node_emb)
    wfp = jnp.pad(W_f, ((0, 6), (0, 0)))
    bf2d = b_f.reshape(1, D)
    b12d = b1.reshape(1, D)

    et = jnp.pad(edges.T.astype(jnp.int32), ((0, 0), (0, E_PAD - E)),
                 constant_values=N).reshape(2, EROWS, CHUNK)
    zeros = jnp.zeros((RPT, H), jnp.float32)

    p2d = lax.dynamic_slice_in_dim(pattern_emb, node_type_id, 1, axis=0)

    ea, eb = _t_init(nodes2d, featp, nembp, wfp, bf2d)
    na, nb = _sc_round(et, ea, eb, zeros)
    ea, eb = _t_round(ea, eb, na, nb, W1, b12d)
    na, nb = _sc_round(et, ea, eb, zeros)
    out = _t_final(ea, eb, na, nb, W1, b12d, p2d)
    return out[:N, 0]


def kernel(nodes, edges, features, node_emb, W_f, b_f, W1, b1, pattern_emb,
           node_type_id):
    return _run(nodes, edges, features, node_emb, W_f, b_f, W1, b1,
                pattern_emb, jnp.asarray(node_type_id, jnp.int32))


# speedup vs baseline: 1.4195x; 1.0241x over previous
import functools

import jax
import jax.numpy as jnp
from jax import lax
from jax.experimental import pallas as pl
from jax.experimental.pallas import tpu as pltpu
from jax.experimental.pallas import tpu_sc as plsc

N = 50000
D = 64
H = 32
BLK = 3584
N_PAD = 50176
E = 800000
CHUNK = 64
NSUB = 16
NCH = 784
IBLK = 56
NPAIR = NCH // 8
E_PAD = NSUB * NCH * CHUNK
EROWS = E_PAD // CHUNK
RPT = N_PAD // NSUB


def _sc_body(et, taba, tabb, zeros, outa, outb,
             sidx, didx, r0, r1, r2, r3, r4, r5, r6, r7,
             acc, g0, g1, g2, g3, g4, g5, g6, g7, ssem):
    srcs = et.at[0]
    dsts = et.at[1]
    c = lax.axis_index("c")
    s = lax.axis_index("s")
    base = s * RPT
    rows = (r0, r1, r2, r3, r4, r5, r6, r7)
    gsems = (g0, g1, g2, g3, g4, g5, g6, g7)

    pltpu.sync_copy(zeros, acc.at[pl.ds(base, RPT)])

    rbase = s * NCH

    plsc.subcore_barrier()

    def drain4():
        pltpu.make_async_copy(zeros.at[pl.ds(0, 4 * CHUNK)],
                              acc.at[pl.ds(0, 4 * CHUNK)], ssem).wait()

    def edge_loop(tab):
        def pair(k, carry):
            @pl.when((k % 7 == 0) & (k > 0))
            def _():
                drain4()
                drain4()

            @pl.when(k % 7 == 0)
            def _():
                ib = rbase + (k // 7) * IBLK
                pltpu.sync_copy(srcs.at[pl.ds(ib, IBLK)], sidx)
                pltpu.sync_copy(dsts.at[pl.ds(ib, IBLK)], didx)

            jb = (k % 7) * 8
            for half in range(2):
                @pl.when(k % 7 != 0)
                def _():
                    drain4()
                hb = half * 4
                cps = [pltpu.async_copy(tab.at[sidx.at[jb + hb + i]],
                                        rows[hb + i], gsems[hb + i])
                       for i in range(4)]
                for i in range(4):
                    cps[i].wait()
                    pltpu.async_copy(rows[hb + i],
                                     acc.at[didx.at[jb + hb + i]], ssem,
                                     add=True)
            return carry
        lax.fori_loop(0, NPAIR, pair, 0)
        drain4()
        drain4()

    @pl.when(c == 0)
    def _():
        edge_loop(taba)

    @pl.when(c == 1)
    def _():
        edge_loop(tabb)

    plsc.subcore_barrier()

    @pl.when(c == 0)
    def _():
        pltpu.sync_copy(acc.at[pl.ds(base, RPT)], outa.at[pl.ds(base, RPT)])

    @pl.when(c == 1)
    def _():
        pltpu.sync_copy(acc.at[pl.ds(base, RPT)], outb.at[pl.ds(base, RPT)])


@functools.cache
def _make_sc_round():
    return functools.partial(
        pl.kernel,
        out_type=(jax.ShapeDtypeStruct((N_PAD, H), jnp.float32),
                  jax.ShapeDtypeStruct((N_PAD, H), jnp.float32)),
        mesh=plsc.VectorSubcoreMesh(core_axis_name="c", subcore_axis_name="s"),
        scratch_types=(
            [pltpu.VMEM((IBLK, CHUNK), jnp.int32)] * 2
            + [pltpu.VMEM((CHUNK, H), jnp.float32)] * 8
            + [pltpu.VMEM_SHARED((N_PAD, H), jnp.float32)]
            + [pltpu.SemaphoreType.DMA] * 9
        ),
        compiler_params=pltpu.CompilerParams(use_tc_tiling_on_sc=False),
    )(_sc_body)


def _sc_round(et, taba, tabb, zeros):
    return _make_sc_round()(et, taba, tabb, zeros)


def _t0_body(nodes_ref, feat_ref, nemb_ref, wf_ref, bf_ref, oa_ref, ob_ref):
    nid = nodes_ref[...]
    iota = lax.broadcasted_iota(jnp.int32, (BLK, 128), 1)
    onehot = (iota == nid).astype(jnp.float32)
    feat = feat_ref[...]
    nemb = nemb_ref[...]
    wf = wf_ref[...]
    bf = bf_ref[...]
    for out, lo in ((oa_ref, 0), (ob_ref, H)):
        emb = jnp.dot(onehot, nemb[:, lo:lo + H],
                      preferred_element_type=jnp.float32)
        fe = jnp.dot(feat, wf[:, lo:lo + H],
                     preferred_element_type=jnp.float32)
        out[...] = emb + fe + bf[:, lo:lo + H]


def _t_init(nodes2d, featp, nembp, wfp, bf2d):
    grid = (N_PAD // BLK,)
    return pl.pallas_call(
        _t0_body,
        grid=grid,
        in_specs=[
            pl.BlockSpec((BLK, 1), lambda i: (i, 0)),
            pl.BlockSpec((BLK, 16), lambda i: (i, 0)),
            pl.BlockSpec((128, D), lambda i: (0, 0)),
            pl.BlockSpec((16, D), lambda i: (0, 0)),
            pl.BlockSpec((1, D), lambda i: (0, 0)),
        ],
        out_specs=(pl.BlockSpec((BLK, H), lambda i: (i, 0)),
                   pl.BlockSpec((BLK, H), lambda i: (i, 0))),
        out_shape=(jax.ShapeDtypeStruct((N_PAD, H), jnp.float32),
                   jax.ShapeDtypeStruct((N_PAD, H), jnp.float32)),
    )(nodes2d, featp, nembp, wfp, bf2d)


def _t1_body(ea_ref, eb_ref, na_ref, nb_ref, w1_ref, b1_ref, oa_ref, ob_ref):
    xa = ea_ref[...] + na_ref[...]
    xb = eb_ref[...] + nb_ref[...]
    w1 = w1_ref[...]
    b1 = b1_ref[...]
    for out, lo in ((oa_ref, 0), (ob_ref, H)):
        h = (jnp.dot(xa, w1[:H, lo:lo + H], preferred_element_type=jnp.float32)
             + jnp.dot(xb, w1[H:, lo:lo + H],
                       preferred_element_type=jnp.float32))
        out[...] = jnp.maximum(h + b1[:, lo:lo + H], 0.0)


def _t_round(ea, eb, na, nb, w1, b12d):
    grid = (N_PAD // BLK,)
    half = pl.BlockSpec((BLK, H), lambda i: (i, 0))
    return pl.pallas_call(
        _t1_body,
        grid=grid,
        in_specs=[half, half, half, half,
                  pl.BlockSpec((D, D), lambda i: (0, 0)),
                  pl.BlockSpec((1, D), lambda i: (0, 0))],
        out_specs=(half, half),
        out_shape=(jax.ShapeDtypeStruct((N_PAD, H), jnp.float32),
                   jax.ShapeDtypeStruct((N_PAD, H), jnp.float32)),
    )(ea, eb, na, nb, w1, b12d)


def _t2_body(ea_ref, eb_ref, na_ref, nb_ref, w1_ref, b1_ref, p_ref, o_ref):
    xa = ea_ref[...] + na_ref[...]
    xb = eb_ref[...] + nb_ref[...]
    w1 = w1_ref[...]
    b1 = b1_ref[...]
    p = p_ref[...]
    num = jnp.zeros((BLK, 1), jnp.float32)
    en2 = jnp.zeros((BLK, 1), jnp.float32)
    for lo in (0, H):
        h = (jnp.dot(xa, w1[:H, lo:lo + H], preferred_element_type=jnp.float32)
             + jnp.dot(xb, w1[H:, lo:lo + H],
                       preferred_element_type=jnp.float32))
        h = jnp.maximum(h + b1[:, lo:lo + H], 0.0)
        num = num + jnp.dot(h, p[:, lo:lo + H].T,
                            preferred_element_type=jnp.float32)
        en2 = en2 + jnp.sum(h * h, axis=1, keepdims=True)
    en = jnp.sqrt(en2)
    pn = jnp.sqrt(jnp.sum(p * p))
    o_ref[...] = num / (jnp.maximum(en, 1e-8) * jnp.maximum(pn, 1e-8))


def _t_final(ea, eb, na, nb, w1, b12d, p2d):
    grid = (N_PAD // BLK,)
    half = pl.BlockSpec((BLK, H), lambda i: (i, 0))
    return pl.pallas_call(
        _t2_body,
        grid=grid,
        in_specs=[half, half, half, half,
                  pl.BlockSpec((D, D), lambda i: (0, 0)),
                  pl.BlockSpec((1, D), lambda i: (0, 0)),
                  pl.BlockSpec((1, D), lambda i: (0, 0))],
        out_specs=pl.BlockSpec((BLK, 1), lambda i: (i, 0)),
        out_shape=jax.ShapeDtypeStruct((N_PAD, 1), jnp.float32),
    )(ea, eb, na, nb, w1, b12d, p2d)


@jax.jit
def _run(nodes, edges, features, node_emb, W_f, b_f, W1, b1, pattern_emb,
         node_type_id):
    nodes2d = jnp.zeros((N_PAD, 1), jnp.int32).at[:N, 0].set(
        nodes.astype(jnp.int32))
    featp = jnp.pad(features, ((0, N_PAD - N), (0, 6)))
    nembp = jnp.zeros((128, D), jnp.float32).at[:100].set(node_emb)
    wfp = jnp.pad(W_f, ((0, 6), (0, 0)))
    bf2d = b_f.reshape(1, D)
    b12d = b1.reshape(1, D)

    et = jnp.pad(edges.T.astype(jnp.int32), ((0, 0), (0, E_PAD - E)),
                 constant_values=N).reshape(2, EROWS, CHUNK)
    zeros = jnp.zeros((RPT, H), jnp.float32)

    p2d = lax.dynamic_slice_in_dim(pattern_emb, node_type_id, 1, axis=0)

    ea, eb = _t_init(nodes2d, featp, nembp, wfp, bf2d)
    na, nb = _sc_round(et, ea, eb, zeros)
    ea, eb = _t_round(ea, eb, na, nb, W1, b12d)
    na, nb = _sc_round(et, ea, eb, zeros)
    out = _t_final(ea, eb, na, nb, W1, b12d, p2d)
    return out[:N, 0]


def kernel(nodes, edges, features, node_emb, W_f, b_f, W1, b1, pattern_emb,
           node_type_id):
    return _run(nodes, edges, features, node_emb, W_f, b_f, W1, b1,
                pattern_emb, jnp.asarray(node_type_id, jnp.int32))


# speedup vs baseline: 1.4232x; 1.0026x over previous
import functools

import jax
import jax.numpy as jnp
from jax import lax
from jax.experimental import pallas as pl
from jax.experimental.pallas import tpu as pltpu
from jax.experimental.pallas import tpu_sc as plsc

N = 50000
D = 64
H = 32
BLK = 7168
N_PAD = 50176
E = 800000
CHUNK = 64
NSUB = 16
NCH = 784
IBLK = 56
NPAIR = NCH // 8
E_PAD = NSUB * NCH * CHUNK
EROWS = E_PAD // CHUNK
RPT = N_PAD // NSUB


def _sc_body(et, taba, tabb, zeros, outa, outb,
             sidx, didx, r0, r1, r2, r3, r4, r5, r6, r7,
             acc, g0, g1, g2, g3, g4, g5, g6, g7, ssem):
    srcs = et.at[0]
    dsts = et.at[1]
    c = lax.axis_index("c")
    s = lax.axis_index("s")
    base = s * RPT
    rows = (r0, r1, r2, r3, r4, r5, r6, r7)
    gsems = (g0, g1, g2, g3, g4, g5, g6, g7)

    pltpu.sync_copy(zeros, acc.at[pl.ds(base, RPT)])

    rbase = s * NCH

    plsc.subcore_barrier()

    def drain4():
        pltpu.make_async_copy(zeros.at[pl.ds(0, 4 * CHUNK)],
                              acc.at[pl.ds(0, 4 * CHUNK)], ssem).wait()

    def edge_loop(tab):
        def pair(k, carry):
            @pl.when((k % 7 == 0) & (k > 0))
            def _():
                drain4()
                drain4()

            @pl.when(k % 7 == 0)
            def _():
                ib = rbase + (k // 7) * IBLK
                pltpu.sync_copy(srcs.at[pl.ds(ib, IBLK)], sidx)
                pltpu.sync_copy(dsts.at[pl.ds(ib, IBLK)], didx)

            jb = (k % 7) * 8
            for half in range(2):
                @pl.when(k % 7 != 0)
                def _():
                    drain4()
                hb = half * 4
                cps = [pltpu.async_copy(tab.at[sidx.at[jb + hb + i]],
                                        rows[hb + i], gsems[hb + i])
                       for i in range(4)]
                for i in range(4):
                    cps[i].wait()
                    pltpu.async_copy(rows[hb + i],
                                     acc.at[didx.at[jb + hb + i]], ssem,
                                     add=True)
            return carry
        lax.fori_loop(0, NPAIR, pair, 0)
        drain4()
        drain4()

    @pl.when(c == 0)
    def _():
        edge_loop(taba)

    @pl.when(c == 1)
    def _():
        edge_loop(tabb)

    plsc.subcore_barrier()

    @pl.when(c == 0)
    def _():
        pltpu.sync_copy(acc.at[pl.ds(base, RPT)], outa.at[pl.ds(base, RPT)])

    @pl.when(c == 1)
    def _():
        pltpu.sync_copy(acc.at[pl.ds(base, RPT)], outb.at[pl.ds(base, RPT)])


@functools.cache
def _make_sc_round():
    return functools.partial(
        pl.kernel,
        out_type=(jax.ShapeDtypeStruct((N_PAD, H), jnp.float32),
                  jax.ShapeDtypeStruct((N_PAD, H), jnp.float32)),
        mesh=plsc.VectorSubcoreMesh(core_axis_name="c", subcore_axis_name="s"),
        scratch_types=(
            [pltpu.VMEM((IBLK, CHUNK), jnp.int32)] * 2
            + [pltpu.VMEM((CHUNK, H), jnp.float32)] * 8
            + [pltpu.VMEM_SHARED((N_PAD, H), jnp.float32)]
            + [pltpu.SemaphoreType.DMA] * 9
        ),
        compiler_params=pltpu.CompilerParams(use_tc_tiling_on_sc=False),
    )(_sc_body)


def _sc_round(et, taba, tabb, zeros):
    return _make_sc_round()(et, taba, tabb, zeros)


def _t0_body(nodes_ref, feat_ref, nemb_ref, wf_ref, bf_ref, oa_ref, ob_ref):
    nid = nodes_ref[...]
    iota = lax.broadcasted_iota(jnp.int32, (BLK, 128), 1)
    onehot = (iota == nid).astype(jnp.float32)
    feat = feat_ref[...]
    nemb = nemb_ref[...]
    wf = wf_ref[...]
    bf = bf_ref[...]
    for out, lo in ((oa_ref, 0), (ob_ref, H)):
        emb = jnp.dot(onehot, nemb[:, lo:lo + H],
                      preferred_element_type=jnp.float32)
        fe = jnp.dot(feat, wf[:, lo:lo + H],
                     preferred_element_type=jnp.float32)
        out[...] = emb + fe + bf[:, lo:lo + H]


def _t_init(nodes2d, featp, nembp, wfp, bf2d):
    grid = (N_PAD // BLK,)
    return pl.pallas_call(
        _t0_body,
        grid=grid,
        in_specs=[
            pl.BlockSpec((BLK, 1), lambda i: (i, 0)),
            pl.BlockSpec((BLK, 16), lambda i: (i, 0)),
            pl.BlockSpec((128, D), lambda i: (0, 0)),
            pl.BlockSpec((16, D), lambda i: (0, 0)),
            pl.BlockSpec((1, D), lambda i: (0, 0)),
        ],
        out_specs=(pl.BlockSpec((BLK, H), lambda i: (i, 0)),
                   pl.BlockSpec((BLK, H), lambda i: (i, 0))),
        out_shape=(jax.ShapeDtypeStruct((N_PAD, H), jnp.float32),
                   jax.ShapeDtypeStruct((N_PAD, H), jnp.float32)),
    )(nodes2d, featp, nembp, wfp, bf2d)


def _t1_body(ea_ref, eb_ref, na_ref, nb_ref, w1_ref, b1_ref, oa_ref, ob_ref):
    xa = ea_ref[...] + na_ref[...]
    xb = eb_ref[...] + nb_ref[...]
    w1 = w1_ref[...]
    b1 = b1_ref[...]
    for out, lo in ((oa_ref, 0), (ob_ref, H)):
        h = (jnp.dot(xa, w1[:H, lo:lo + H], preferred_element_type=jnp.float32)
             + jnp.dot(xb, w1[H:, lo:lo + H],
                       preferred_element_type=jnp.float32))
        out[...] = jnp.maximum(h + b1[:, lo:lo + H], 0.0)


def _t_round(ea, eb, na, nb, w1, b12d):
    grid = (N_PAD // BLK,)
    half = pl.BlockSpec((BLK, H), lambda i: (i, 0))
    return pl.pallas_call(
        _t1_body,
        grid=grid,
        in_specs=[half, half, half, half,
                  pl.BlockSpec((D, D), lambda i: (0, 0)),
                  pl.BlockSpec((1, D), lambda i: (0, 0))],
        out_specs=(half, half),
        out_shape=(jax.ShapeDtypeStruct((N_PAD, H), jnp.float32),
                   jax.ShapeDtypeStruct((N_PAD, H), jnp.float32)),
    )(ea, eb, na, nb, w1, b12d)


def _t2_body(ea_ref, eb_ref, na_ref, nb_ref, w1_ref, b1_ref, p_ref, o_ref):
    xa = ea_ref[...] + na_ref[...]
    xb = eb_ref[...] + nb_ref[...]
    w1 = w1_ref[...]
    b1 = b1_ref[...]
    p = p_ref[...]
    num = jnp.zeros((BLK, 1), jnp.float32)
    en2 = jnp.zeros((BLK, 1), jnp.float32)
    for lo in (0, H):
        h = (jnp.dot(xa, w1[:H, lo:lo + H], preferred_element_type=jnp.float32)
             + jnp.dot(xb, w1[H:, lo:lo + H],
                       preferred_element_type=jnp.float32))
        h = jnp.maximum(h + b1[:, lo:lo + H], 0.0)
        num = num + jnp.dot(h, p[:, lo:lo + H].T,
                            preferred_element_type=jnp.float32)
        en2 = en2 + jnp.sum(h * h, axis=1, keepdims=True)
    en = jnp.sqrt(en2)
    pn = jnp.sqrt(jnp.sum(p * p))
    o_ref[...] = num / (jnp.maximum(en, 1e-8) * jnp.maximum(pn, 1e-8))


def _t_final(ea, eb, na, nb, w1, b12d, p2d):
    grid = (N_PAD // BLK,)
    half = pl.BlockSpec((BLK, H), lambda i: (i, 0))
    return pl.pallas_call(
        _t2_body,
        grid=grid,
        in_specs=[half, half, half, half,
                  pl.BlockSpec((D, D), lambda i: (0, 0)),
                  pl.BlockSpec((1, D), lambda i: (0, 0)),
                  pl.BlockSpec((1, D), lambda i: (0, 0))],
        out_specs=pl.BlockSpec((BLK, 1), lambda i: (i, 0)),
        out_shape=jax.ShapeDtypeStruct((N_PAD, 1), jnp.float32),
    )(ea, eb, na, nb, w1, b12d, p2d)


@jax.jit
def _run(nodes, edges, features, node_emb, W_f, b_f, W1, b1, pattern_emb,
         node_type_id):
    nodes2d = jnp.zeros((N_PAD, 1), jnp.int32).at[:N, 0].set(
        nodes.astype(jnp.int32))
    featp = jnp.pad(features, ((0, N_PAD - N), (0, 6)))
    nembp = jnp.zeros((128, D), jnp.float32).at[:100].set(node_emb)
    wfp = jnp.pad(W_f, ((0, 6), (0, 0)))
    bf2d = b_f.reshape(1, D)
    b12d = b1.reshape(1, D)

    et = jnp.pad(edges.T.astype(jnp.int32), ((0, 0), (0, E_PAD - E)),
                 constant_values=N).reshape(2, EROWS, CHUNK)
    zeros = jnp.zeros((RPT, H), jnp.float32)

    p2d = lax.dynamic_slice_in_dim(pattern_emb, node_type_id, 1, axis=0)

    ea, eb = _t_init(nodes2d, featp, nembp, wfp, bf2d)
    na, nb = _sc_round(et, ea, eb, zeros)
    ea, eb = _t_round(ea, eb, na, nb, W1, b12d)
    na, nb = _sc_round(et, ea, eb, zeros)
    out = _t_final(ea, eb, na, nb, W1, b12d, p2d)
    return out[:N, 0]


def kernel(nodes, edges, features, node_emb, W_f, b_f, W1, b1, pattern_emb,
           node_type_id):
    return _run(nodes, edges, features, node_emb, W_f, b_f, W1, b1,
                pattern_emb, jnp.asarray(node_type_id, jnp.int32))
